# Initial kernel scaffold; baseline (speedup 1.0000x reference)
import functools

import jax
import jax.numpy as jnp
from jax import lax
from jax.experimental import pallas as pl
from jax.experimental.pallas import tpu as pltpu
from jax.experimental.pallas import tpu_sc as plsc

N = 10000
E = 320000
D = 128
G = 64

NC = 2
NS = 16
NW = NC * NS

EPW = E // NW
CHUNK = 80
NITER = EPW // CHUNK

ROWS_PER_TILE = N // NS
ZCH = 125
DEG_PAD = 10240
DEG_PER_TILE = DEG_PAD // NS

TCB = 400
TCG = N // TCB

_mesh = plsc.VectorSubcoreMesh(core_axis_name="c", subcore_axis_name="s")


@functools.partial(
    pl.kernel,
    out_type=jax.ShapeDtypeStruct((2 * DEG_PAD,), jnp.float32),
    mesh=_mesh,
    scratch_types=[
        pltpu.VMEM((CHUNK,), jnp.int32),
        pltpu.VMEM((CHUNK,), jnp.float32),
        pltpu.VMEM((DEG_PER_TILE,), jnp.float32),
        pltpu.VMEM_SHARED((DEG_PAD,), jnp.float32),
    ],
)
def _deg_kernel(dst_hbm, out_hbm, dstv, onesv, zv, deg_sp):
    c = lax.axis_index("c")
    s = lax.axis_index("s")

    one = jnp.full((16,), 1.0, jnp.float32)
    zero = jnp.zeros((16,), jnp.float32)

    def fill(i, _):
        onesv[pl.ds(i * 16, 16)] = one
        return 0
    lax.fori_loop(0, CHUNK // 16, fill, 0)

    def zfill(i, _):
        zv[pl.ds(i * 16, 16)] = zero
        return 0
    lax.fori_loop(0, DEG_PER_TILE // 16, zfill, 0)

    pltpu.sync_copy(zv, deg_sp.at[pl.ds(s * DEG_PER_TILE, DEG_PER_TILE)])
    plsc.subcore_barrier()

    base0 = (c * NS + s) * EPW

    def body(i, _):
        pltpu.sync_copy(dst_hbm.at[pl.ds(base0 + i * CHUNK, CHUNK)], dstv)
        pltpu.sync_copy(onesv, deg_sp.at[dstv], add=True)
        return 0
    lax.fori_loop(0, NITER, body, 0)

    plsc.subcore_barrier()
    pltpu.sync_copy(
        deg_sp.at[pl.ds(s * DEG_PER_TILE, DEG_PER_TILE)],
        out_hbm.at[pl.ds(c * DEG_PAD + s * DEG_PER_TILE, DEG_PER_TILE)],
    )


@functools.partial(
    pl.kernel,
    out_type=jax.ShapeDtypeStruct((2 * N, D), jnp.float32),
    mesh=_mesh,
    scratch_types=[
        pltpu.VMEM((CHUNK,), jnp.int32),
        pltpu.VMEM((CHUNK,), jnp.int32),
        pltpu.VMEM((CHUNK, D), jnp.float32),
        pltpu.VMEM((ZCH, D), jnp.float32),
        pltpu.VMEM_SHARED((N, D), jnp.float32),
        pltpu.SemaphoreType.DMA,
    ],
)
def _agg_kernel(table_hbm, src_hbm, dst_hbm, out_hbm,
                srcv, dstv, rows, zbuf, acc, sem):
    c = lax.axis_index("c")
    s = lax.axis_index("s")

    zero = jnp.zeros((16,), jnp.float32)

    def zfill(i, _):
        for j in range(D // 16):
            zbuf[i, pl.ds(j * 16, 16)] = zero
        return 0
    lax.fori_loop(0, ZCH, zfill, 0)

    r0 = s * ROWS_PER_TILE
    for j in range(ROWS_PER_TILE // ZCH):
        pltpu.sync_copy(zbuf, acc.at[pl.ds(r0 + j * ZCH, ZCH)])
    plsc.subcore_barrier()

    base0 = (c * NS + s) * EPW

    def body(i, _):
        base = base0 + i * CHUNK
        pltpu.sync_copy(src_hbm.at[pl.ds(base, CHUNK)], srcv)
        pltpu.sync_copy(dst_hbm.at[pl.ds(base, CHUNK)], dstv)
        pltpu.async_copy(table_hbm.at[srcv], rows, sem).wait()
        pltpu.sync_copy(rows, acc.at[dstv], add=True)
        return 0
    lax.fori_loop(0, NITER, body, 0)

    plsc.subcore_barrier()
    pltpu.sync_copy(
        acc.at[pl.ds(r0, ROWS_PER_TILE)],
        out_hbm.at[pl.ds(c * N + r0, ROWS_PER_TILE)],
    )


def _tc1_body(x_ref, w_ref, d0_ref, d1_ref, xs_ref, dinv_ref):
    deg = d0_ref[...] + d1_ref[...] + 1.0
    dv = lax.rsqrt(deg)
    xw = jnp.dot(x_ref[...], w_ref[...], preferred_element_type=jnp.float32)
    xs_ref[...] = dv * xw
    dinv_ref[...] = dv


def _tc2_body(a0_ref, a1_ref, xs1_ref, dinv_ref, w_ref, b_ref, xs2_ref):
    dv = dinv_ref[...]
    h = dv * (a0_ref[...] + a1_ref[...] + xs1_ref[...]) + b_ref[...]
    h = jnp.maximum(h, 0.0)
    xs2_ref[...] = dv * jnp.dot(h, w_ref[...],
                                preferred_element_type=jnp.float32)


def _tc3_body(a0_ref, a1_ref, xs2_ref, dinv_ref, b_ref, batch_ref,
              pool_ref, cnt_ref):
    i = pl.program_id(0)
    dv = dinv_ref[...]
    out = dv * (a0_ref[...] + a1_ref[...] + xs2_ref[...]) + b_ref[...]

    b_row = batch_ref[0]
    gids = lax.broadcasted_iota(jnp.int32, (G, TCB), 0)
    oht = (gids == b_row).astype(jnp.float32)

    ps = jnp.dot(oht, out, preferred_element_type=jnp.float32)
    cs = jnp.dot(oht, jnp.ones_like(out), preferred_element_type=jnp.float32)

    @pl.when(i == 0)
    def _():
        pool_ref[...] = jnp.zeros_like(pool_ref)
        cnt_ref[...] = jnp.zeros_like(cnt_ref)

    pool_ref[...] += ps
    cnt_ref[...] += cs

    @pl.when(i == TCG - 1)
    def _():
        pool_ref[...] = pool_ref[...] / jnp.maximum(cnt_ref[...], 1.0)


def _rowspec():
    return pl.BlockSpec((TCB, D), lambda i: (i, 0))


def _colspec():
    return pl.BlockSpec((TCB, 1), lambda i: (i, 0))


def _fullspec(shape):
    nd = len(shape)
    return pl.BlockSpec(shape, lambda i: (0,) * nd)


def kernel(x, edge_index, batch, W1, b1, W2, b2):
    src = edge_index[0]
    dst = edge_index[1]

    deg2 = _deg_kernel(dst)
    d0 = deg2[:N].reshape(N, 1)
    d1 = deg2[DEG_PAD:DEG_PAD + N].reshape(N, 1)

    b1r = b1.reshape(1, D)
    b2r = b2.reshape(1, D)
    batch3 = batch.reshape(TCG, 1, TCB)

    xs1, dinv = pl.pallas_call(
        _tc1_body,
        grid=(TCG,),
        in_specs=[_rowspec(), _fullspec((D, D)), _colspec(), _colspec()],
        out_specs=[_rowspec(), _colspec()],
        out_shape=[
            jax.ShapeDtypeStruct((N, D), jnp.float32),
            jax.ShapeDtypeStruct((N, 1), jnp.float32),
        ],
    )(x, W1, d0, d1)

    agg1 = _agg_kernel(xs1, src, dst)

    xs2 = pl.pallas_call(
        _tc2_body,
        grid=(TCG,),
        in_specs=[_rowspec(), _rowspec(), _rowspec(), _colspec(),
                  _fullspec((D, D)), _fullspec((1, D))],
        out_specs=_rowspec(),
        out_shape=jax.ShapeDtypeStruct((N, D), jnp.float32),
    )(agg1[:N], agg1[N:], xs1, dinv, W2, b1r)

    agg2 = _agg_kernel(xs2, src, dst)

    pooled = pl.pallas_call(
        _tc3_body,
        grid=(TCG,),
        in_specs=[_rowspec(), _rowspec(), _rowspec(), _colspec(),
                  _fullspec((1, D)),
                  pl.BlockSpec((1, 1, TCB), lambda i: (i, 0, 0))],
        out_specs=pl.BlockSpec((G, D), lambda i: (0, 0)),
        out_shape=jax.ShapeDtypeStruct((G, D), jnp.float32),
        scratch_shapes=[pltpu.VMEM((G, D), jnp.float32)],
    )(agg2[:N], agg2[N:], xs2, dinv, b2r, batch3)

    return pooled


# speedup vs baseline: 12.9433x; 12.9433x over previous
import functools

import jax
import jax.numpy as jnp
from jax import lax
from jax.experimental import pallas as pl
from jax.experimental.pallas import tpu as pltpu
from jax.experimental.pallas import tpu_sc as plsc

N = 10000
E = 320000
D = 128
G = 64

NC = 2
NS = 16
NW = NC * NS

EPW = E // NW
CHUNK = 80
NITER = EPW // CHUNK

NPAD = 10240
ROWS_PER_TILE = NPAD // NS
ZCH = 128
DEG_PAD = NPAD
DEG_PER_TILE = DEG_PAD // NS

TCB = 400
TCG = N // TCB

_mesh = plsc.VectorSubcoreMesh(core_axis_name="c", subcore_axis_name="s")


@functools.partial(
    pl.kernel,
    out_type=jax.ShapeDtypeStruct((2 * DEG_PAD,), jnp.float32),
    mesh=_mesh,
    scratch_types=[
        pltpu.VMEM((CHUNK,), jnp.int32),
        pltpu.VMEM((CHUNK,), jnp.float32),
        pltpu.VMEM((DEG_PER_TILE,), jnp.float32),
        pltpu.VMEM_SHARED((DEG_PAD,), jnp.float32),
    ],
)
def _deg_kernel(dst_hbm, out_hbm, dstv, onesv, zv, deg_sp):
    c = lax.axis_index("c")
    s = lax.axis_index("s")

    one = jnp.full((16,), 1.0, jnp.float32)
    zero = jnp.zeros((16,), jnp.float32)

    def fill(i, _):
        onesv[pl.ds(i * 16, 16)] = one
        return 0
    lax.fori_loop(0, CHUNK // 16, fill, 0)

    def zfill(i, _):
        zv[pl.ds(i * 16, 16)] = zero
        return 0
    lax.fori_loop(0, DEG_PER_TILE // 16, zfill, 0)

    pltpu.sync_copy(zv, deg_sp.at[pl.ds(s * DEG_PER_TILE, DEG_PER_TILE)])
    plsc.subcore_barrier()

    base0 = (c * NS + s) * EPW

    def body(i, _):
        pltpu.sync_copy(dst_hbm.at[pl.ds(base0 + i * CHUNK, CHUNK)], dstv)
        pltpu.sync_copy(onesv, deg_sp.at[dstv], add=True)
        return 0
    lax.fori_loop(0, NITER, body, 0)

    plsc.subcore_barrier()
    pltpu.sync_copy(
        deg_sp.at[pl.ds(s * DEG_PER_TILE, DEG_PER_TILE)],
        out_hbm.at[pl.ds(c * DEG_PAD + s * DEG_PER_TILE, DEG_PER_TILE)],
    )


@functools.partial(
    pl.kernel,
    out_type=jax.ShapeDtypeStruct((2 * NPAD, D), jnp.float32),
    mesh=_mesh,
    scratch_types=[
        pltpu.VMEM((CHUNK,), jnp.int32),
        pltpu.VMEM((CHUNK,), jnp.int32),
        pltpu.VMEM((CHUNK, D), jnp.float32),
        pltpu.VMEM((ZCH, D), jnp.float32),
        pltpu.VMEM_SHARED((NPAD, D), jnp.float32),
        pltpu.SemaphoreType.DMA,
    ],
)
def _agg_kernel(table_hbm, src_hbm, dst_hbm, out_hbm,
                srcv, dstv, rows, zbuf, acc, sem):
    c = lax.axis_index("c")
    s = lax.axis_index("s")

    zero = jnp.zeros((16,), jnp.float32)

    def zfill(i, _):
        for j in range(D // 16):
            zbuf[i, pl.ds(j * 16, 16)] = zero
        return 0
    lax.fori_loop(0, ZCH, zfill, 0)

    r0 = s * ROWS_PER_TILE
    for j in range(ROWS_PER_TILE // ZCH):
        pltpu.sync_copy(zbuf, acc.at[pl.ds(r0 + j * ZCH, ZCH)])
    plsc.subcore_barrier()

    base0 = (c * NS + s) * EPW

    def body(i, _):
        base = base0 + i * CHUNK
        pltpu.sync_copy(src_hbm.at[pl.ds(base, CHUNK)], srcv)
        pltpu.sync_copy(dst_hbm.at[pl.ds(base, CHUNK)], dstv)
        pltpu.async_copy(table_hbm.at[srcv], rows, sem).wait()
        pltpu.sync_copy(rows, acc.at[dstv], add=True)
        return 0
    lax.fori_loop(0, NITER, body, 0)

    plsc.subcore_barrier()
    pltpu.sync_copy(
        acc.at[pl.ds(r0, ROWS_PER_TILE)],
        out_hbm.at[pl.ds(c * NPAD + r0, ROWS_PER_TILE)],
    )


def _tc1_body(x_ref, w_ref, d0_ref, d1_ref, xs_ref, dinv_ref):
    deg = d0_ref[...] + d1_ref[...] + 1.0
    dv = lax.rsqrt(deg)
    xw = jnp.dot(x_ref[...], w_ref[...], preferred_element_type=jnp.float32)
    xs_ref[...] = dv * xw
    dinv_ref[...] = dv


def _tc2_body(a0_ref, a1_ref, xs1_ref, dinv_ref, w_ref, b_ref, xs2_ref):
    dv = dinv_ref[...]
    h = dv * (a0_ref[...] + a1_ref[...] + xs1_ref[...]) + b_ref[...]
    h = jnp.maximum(h, 0.0)
    xs2_ref[...] = dv * jnp.dot(h, w_ref[...],
                                preferred_element_type=jnp.float32)


def _tc3_body(a0_ref, a1_ref, xs2_ref, dinv_ref, b_ref, batch_ref,
              pool_ref, cnt_ref):
    i = pl.program_id(0)
    dv = dinv_ref[...]
    out = dv * (a0_ref[...] + a1_ref[...] + xs2_ref[...]) + b_ref[...]

    b_row = batch_ref[0]
    gids = lax.broadcasted_iota(jnp.int32, (G, TCB), 0)
    oht = (gids == b_row).astype(jnp.float32)

    ps = jnp.dot(oht, out, preferred_element_type=jnp.float32)
    cs = jnp.dot(oht, jnp.ones_like(out), preferred_element_type=jnp.float32)

    @pl.when(i == 0)
    def _():
        pool_ref[...] = jnp.zeros_like(pool_ref)
        cnt_ref[...] = jnp.zeros_like(cnt_ref)

    pool_ref[...] += ps
    cnt_ref[...] += cs

    @pl.when(i == TCG - 1)
    def _():
        pool_ref[...] = pool_ref[...] / jnp.maximum(cnt_ref[...], 1.0)


def _rowspec():
    return pl.BlockSpec((TCB, D), lambda i: (i, 0))


def _colspec():
    return pl.BlockSpec((TCB, 1), lambda i: (i, 0))


def _fullspec(shape):
    nd = len(shape)
    return pl.BlockSpec(shape, lambda i: (0,) * nd)


def kernel(x, edge_index, batch, W1, b1, W2, b2):
    src = edge_index[0]
    dst = edge_index[1]

    deg2 = _deg_kernel(dst)
    d0 = deg2[:N].reshape(N, 1)
    d1 = deg2[DEG_PAD:DEG_PAD + N].reshape(N, 1)

    b1r = b1.reshape(1, D)
    b2r = b2.reshape(1, D)
    batch3 = batch.reshape(TCG, 1, TCB)

    xs1, dinv = pl.pallas_call(
        _tc1_body,
        grid=(TCG,),
        in_specs=[_rowspec(), _fullspec((D, D)), _colspec(), _colspec()],
        out_specs=[_rowspec(), _colspec()],
        out_shape=[
            jax.ShapeDtypeStruct((N, D), jnp.float32),
            jax.ShapeDtypeStruct((N, 1), jnp.float32),
        ],
    )(x, W1, d0, d1)

    agg1 = _agg_kernel(xs1, src, dst)

    xs2 = pl.pallas_call(
        _tc2_body,
        grid=(TCG,),
        in_specs=[_rowspec(), _rowspec(), _rowspec(), _colspec(),
                  _fullspec((D, D)), _fullspec((1, D))],
        out_specs=_rowspec(),
        out_shape=jax.ShapeDtypeStruct((N, D), jnp.float32),
    )(agg1[:N], agg1[NPAD:NPAD + N], xs1, dinv, W2, b1r)

    agg2 = _agg_kernel(xs2, src, dst)

    pooled = pl.pallas_call(
        _tc3_body,
        grid=(TCG,),
        in_specs=[_rowspec(), _rowspec(), _rowspec(), _colspec(),
                  _fullspec((1, D)),
                  pl.BlockSpec((1, 1, TCB), lambda i: (i, 0, 0))],
        out_specs=pl.BlockSpec((G, D), lambda i: (0, 0)),
        out_shape=jax.ShapeDtypeStruct((G, D), jnp.float32),
        scratch_shapes=[pltpu.VMEM((G, D), jnp.float32)],
    )(agg2[:N], agg2[NPAD:NPAD + N], xs2, dinv, b2r, batch3)

    return pooled


# speedup vs baseline: 20.8679x; 1.6123x over previous
import functools

import jax
import jax.numpy as jnp
from jax import lax
from jax.experimental import pallas as pl
from jax.experimental.pallas import tpu as pltpu
from jax.experimental.pallas import tpu_sc as plsc

N = 10000
E = 320000
D = 128
G = 64

NC = 2
NS = 16
NW = NC * NS

EPW = E // NW
CHUNK = 80
NITER = EPW // CHUNK

NPAD = 10240
ROWS_PER_TILE = NPAD // NS
ZCH = 128
DEG_PAD = NPAD
DEG_PER_TILE = DEG_PAD // NS

TCB = 400
TCG = N // TCB

_mesh = plsc.VectorSubcoreMesh(core_axis_name="c", subcore_axis_name="s")


@functools.partial(
    pl.kernel,
    out_type=jax.ShapeDtypeStruct((2 * DEG_PAD,), jnp.float32),
    mesh=_mesh,
    scratch_types=[
        pltpu.VMEM((CHUNK,), jnp.int32),
        pltpu.VMEM((CHUNK,), jnp.float32),
        pltpu.VMEM((DEG_PER_TILE,), jnp.float32),
        pltpu.VMEM_SHARED((DEG_PAD,), jnp.float32),
    ],
)
def _deg_kernel(dst_hbm, out_hbm, dstv, onesv, zv, deg_sp):
    c = lax.axis_index("c")
    s = lax.axis_index("s")

    one = jnp.full((16,), 1.0, jnp.float32)
    zero = jnp.zeros((16,), jnp.float32)

    def fill(i, _):
        onesv[pl.ds(i * 16, 16)] = one
        return 0
    lax.fori_loop(0, CHUNK // 16, fill, 0)

    def zfill(i, _):
        zv[pl.ds(i * 16, 16)] = zero
        return 0
    lax.fori_loop(0, DEG_PER_TILE // 16, zfill, 0)

    pltpu.sync_copy(zv, deg_sp.at[pl.ds(s * DEG_PER_TILE, DEG_PER_TILE)])
    plsc.subcore_barrier()

    base0 = (c * NS + s) * EPW

    def body(i, _):
        pltpu.sync_copy(dst_hbm.at[pl.ds(base0 + i * CHUNK, CHUNK)], dstv)
        pltpu.sync_copy(onesv, deg_sp.at[dstv], add=True)
        return 0
    lax.fori_loop(0, NITER, body, 0)

    plsc.subcore_barrier()
    pltpu.sync_copy(
        deg_sp.at[pl.ds(s * DEG_PER_TILE, DEG_PER_TILE)],
        out_hbm.at[pl.ds(c * DEG_PAD + s * DEG_PER_TILE, DEG_PER_TILE)],
    )


@functools.partial(
    pl.kernel,
    out_type=jax.ShapeDtypeStruct((2 * NPAD, D), jnp.float32),
    mesh=_mesh,
    scratch_types=[
        pltpu.VMEM((EPW,), jnp.int32),
        pltpu.VMEM((NITER, CHUNK), jnp.int32),
        pltpu.VMEM((CHUNK, D), jnp.float32),
        pltpu.VMEM((CHUNK, D), jnp.float32),
        pltpu.VMEM_SHARED((NPAD, D), jnp.float32),
        pltpu.SemaphoreType.DMA,
        pltpu.SemaphoreType.DMA,
    ],
)
def _agg_kernel(table_hbm, src_hbm, dst2_hbm, out_hbm,
                srcv, dstv, rows0, rows1, acc, sem0, sem1):
    c = lax.axis_index("c")
    s = lax.axis_index("s")
    w = c * NS + s

    zero = jnp.zeros((16,), jnp.float32)

    def zfill(i, _):
        for j in range(D // 16):
            rows0[i, pl.ds(j * 16, 16)] = zero
        return 0
    lax.fori_loop(0, CHUNK, zfill, 0)

    pltpu.sync_copy(src_hbm.at[pl.ds(w * EPW, EPW)], srcv)
    pltpu.sync_copy(dst2_hbm.at[w], dstv)

    r0 = s * ROWS_PER_TILE
    for j in range(ROWS_PER_TILE // CHUNK):
        pltpu.sync_copy(rows0, acc.at[pl.ds(r0 + j * CHUNK, CHUNK)])
    plsc.subcore_barrier()

    rows = (rows0, rows1)
    sems = (sem0, sem1)

    def gsrc(ie):
        return table_hbm.at[srcv.at[pl.ds(ie * CHUNK, CHUNK)]]

    pltpu.async_copy(gsrc(0), rows0, sem0)

    def body(i, _):
        for b in range(2):
            ie = 2 * i + b
            pltpu.make_async_copy(gsrc(ie), rows[b], sems[b]).wait()
            pltpu.async_copy(gsrc(ie + 1), rows[1 - b], sems[1 - b])
            pltpu.sync_copy(rows[b], acc.at[dstv.at[ie]], add=True)
        return 0
    lax.fori_loop(0, (NITER - 1) // 2, body, 0)

    last = NITER - 1
    pltpu.make_async_copy(gsrc(last), rows0, sem0).wait()
    pltpu.sync_copy(rows0, acc.at[dstv.at[last]], add=True)

    plsc.subcore_barrier()
    pltpu.sync_copy(
        acc.at[pl.ds(r0, ROWS_PER_TILE)],
        out_hbm.at[pl.ds(c * NPAD + r0, ROWS_PER_TILE)],
    )


def _tc1_body(x_ref, w_ref, d0_ref, d1_ref, xs_ref, dinv_ref):
    deg = d0_ref[...] + d1_ref[...] + 1.0
    dv = lax.rsqrt(deg)
    xw = jnp.dot(x_ref[...], w_ref[...], preferred_element_type=jnp.float32)
    xs_ref[...] = dv * xw
    dinv_ref[...] = dv


def _tc2_body(a0_ref, a1_ref, xs1_ref, dinv_ref, w_ref, b_ref, xs2_ref):
    dv = dinv_ref[...]
    h = dv * (a0_ref[...] + a1_ref[...] + xs1_ref[...]) + b_ref[...]
    h = jnp.maximum(h, 0.0)
    xs2_ref[...] = dv * jnp.dot(h, w_ref[...],
                                preferred_element_type=jnp.float32)


def _tc3_body(a0_ref, a1_ref, xs2_ref, dinv_ref, b_ref, batch_ref,
              pool_ref, cnt_ref):
    i = pl.program_id(0)
    dv = dinv_ref[...]
    out = dv * (a0_ref[...] + a1_ref[...] + xs2_ref[...]) + b_ref[...]

    b_row = batch_ref[0]
    gids = lax.broadcasted_iota(jnp.int32, (G, TCB), 0)
    oht = (gids == b_row).astype(jnp.float32)

    ps = jnp.dot(oht, out, preferred_element_type=jnp.float32)
    cs = jnp.dot(oht, jnp.ones_like(out), preferred_element_type=jnp.float32)

    @pl.when(i == 0)
    def _():
        pool_ref[...] = jnp.zeros_like(pool_ref)
        cnt_ref[...] = jnp.zeros_like(cnt_ref)

    pool_ref[...] += ps
    cnt_ref[...] += cs

    @pl.when(i == TCG - 1)
    def _():
        pool_ref[...] = pool_ref[...] / jnp.maximum(cnt_ref[...], 1.0)


def _rowspec():
    return pl.BlockSpec((TCB, D), lambda i: (i, 0))


def _colspec():
    return pl.BlockSpec((TCB, 1), lambda i: (i, 0))


def _fullspec(shape):
    nd = len(shape)
    return pl.BlockSpec(shape, lambda i: (0,) * nd)


def kernel(x, edge_index, batch, W1, b1, W2, b2):
    src = edge_index[0]
    dst = edge_index[1]
    dst2 = dst.reshape(NW, NITER, CHUNK)

    deg2 = _deg_kernel(dst)
    d0 = deg2[:N].reshape(N, 1)
    d1 = deg2[DEG_PAD:DEG_PAD + N].reshape(N, 1)

    b1r = b1.reshape(1, D)
    b2r = b2.reshape(1, D)
    batch3 = batch.reshape(TCG, 1, TCB)

    xs1, dinv = pl.pallas_call(
        _tc1_body,
        grid=(TCG,),
        in_specs=[_rowspec(), _fullspec((D, D)), _colspec(), _colspec()],
        out_specs=[_rowspec(), _colspec()],
        out_shape=[
            jax.ShapeDtypeStruct((N, D), jnp.float32),
            jax.ShapeDtypeStruct((N, 1), jnp.float32),
        ],
    )(x, W1, d0, d1)

    agg1 = _agg_kernel(xs1, src, dst2)

    xs2 = pl.pallas_call(
        _tc2_body,
        grid=(TCG,),
        in_specs=[_rowspec(), _rowspec(), _rowspec(), _colspec(),
                  _fullspec((D, D)), _fullspec((1, D))],
        out_specs=_rowspec(),
        out_shape=jax.ShapeDtypeStruct((N, D), jnp.float32),
    )(agg1[:N], agg1[NPAD:NPAD + N], xs1, dinv, W2, b1r)

    agg2 = _agg_kernel(xs2, src, dst2)

    pooled = pl.pallas_call(
        _tc3_body,
        grid=(TCG,),
        in_specs=[_rowspec(), _rowspec(), _rowspec(), _colspec(),
                  _fullspec((1, D)),
                  pl.BlockSpec((1, 1, TCB), lambda i: (i, 0, 0))],
        out_specs=pl.BlockSpec((G, D), lambda i: (0, 0)),
        out_shape=jax.ShapeDtypeStruct((G, D), jnp.float32),
        scratch_shapes=[pltpu.VMEM((G, D), jnp.float32)],
    )(agg2[:N], agg2[NPAD:NPAD + N], xs2, dinv, b2r, batch3)

    return pooled


# speedup vs baseline: 32.6121x; 1.5628x over previous
import functools

import jax
import jax.numpy as jnp
from jax import lax
from jax.experimental import pallas as pl
from jax.experimental.pallas import tpu as pltpu
from jax.experimental.pallas import tpu_sc as plsc

N = 10000
E = 320000
D = 128
G = 64

NC = 2
NS = 16
NW = NC * NS

EPW = E // NW
CHUNK = 80
NITER = EPW // CHUNK

NPAD = 10240
ROWS_PER_TILE = NPAD // NS
ZCH = 128
DEG_PAD = NPAD
DEG_PER_TILE = DEG_PAD // NS

TCB = 400
TCG = N // TCB

_mesh = plsc.VectorSubcoreMesh(core_axis_name="c", subcore_axis_name="s")


@functools.partial(
    pl.kernel,
    out_type=jax.ShapeDtypeStruct((2 * DEG_PAD,), jnp.float32),
    mesh=_mesh,
    compiler_params=pltpu.CompilerParams(needs_layout_passes=False),
    scratch_types=[
        pltpu.VMEM((EPW,), jnp.int32),
        pltpu.VMEM((DEG_PAD,), jnp.float32),
        pltpu.VMEM((NS, DEG_PER_TILE), jnp.float32),
        pltpu.VMEM((DEG_PER_TILE,), jnp.float32),
        pltpu.VMEM_SHARED((NS, DEG_PAD), jnp.float32),
    ],
)
def _deg_kernel(dst_hbm, out_hbm, dstv, hist, segs, degv, hist_sp):
    c = lax.axis_index("c")
    s = lax.axis_index("s")
    w = c * NS + s

    one = jnp.full((16,), 1.0, jnp.float32)
    zero = jnp.zeros((16,), jnp.float32)

    def zfill(i, _):
        hist[pl.ds(i * 16, 16)] = zero
        return 0
    lax.fori_loop(0, DEG_PAD // 16, zfill, 0)

    pltpu.sync_copy(dst_hbm.at[pl.ds(w * EPW, EPW)], dstv)

    def hbody(i, _):
        idx = dstv[pl.ds(i * 16, 16)]
        plsc.addupdate_scatter(hist, [idx], one)
        return 0
    lax.fori_loop(0, EPW // 16, hbody, 0)

    pltpu.sync_copy(hist, hist_sp.at[s])
    plsc.subcore_barrier()
    pltpu.sync_copy(
        hist_sp.at[pl.ds(0, NS), pl.ds(s * DEG_PER_TILE, DEG_PER_TILE)],
        segs)

    def sbody(j, _):
        a = segs[0, pl.ds(j * 16, 16)]
        for t in range(1, NS):
            a = a + segs[t, pl.ds(j * 16, 16)]
        degv[pl.ds(j * 16, 16)] = a
        return 0
    lax.fori_loop(0, DEG_PER_TILE // 16, sbody, 0)

    pltpu.sync_copy(
        degv,
        out_hbm.at[pl.ds(c * DEG_PAD + s * DEG_PER_TILE, DEG_PER_TILE)],
    )


@functools.partial(
    pl.kernel,
    out_type=jax.ShapeDtypeStruct((2 * NPAD, D), jnp.float32),
    mesh=_mesh,
    scratch_types=[
        pltpu.VMEM((EPW,), jnp.int32),
        pltpu.VMEM((CHUNK,), jnp.int32),
        pltpu.VMEM((CHUNK,), jnp.int32),
        pltpu.VMEM((CHUNK,), jnp.int32),
        pltpu.VMEM((CHUNK, D), jnp.float32),
        pltpu.VMEM((CHUNK, D), jnp.float32),
        pltpu.VMEM((CHUNK, D), jnp.float32),
        pltpu.VMEM_SHARED((NPAD, D), jnp.float32),
        pltpu.SemaphoreType.DMA,
        pltpu.SemaphoreType.DMA,
        pltpu.SemaphoreType.DMA,
        pltpu.SemaphoreType.DMA,
        pltpu.SemaphoreType.DMA,
        pltpu.SemaphoreType.DMA,
    ],
)
def _agg_kernel(table_hbm, src_hbm, dst_hbm, out_hbm,
                srcv, db0, db1, db2, rows0, rows1, rows2, acc,
                gs0, gs1, gs2, ds0, ds1, ds2):
    c = lax.axis_index("c")
    s = lax.axis_index("s")
    w = c * NS + s

    zero = jnp.zeros((16,), jnp.float32)

    def zfill(i, _):
        for j in range(D // 16):
            rows0[i, pl.ds(j * 16, 16)] = zero
        return 0
    lax.fori_loop(0, CHUNK, zfill, 0)

    pltpu.sync_copy(src_hbm.at[pl.ds(w * EPW, EPW)], srcv)

    r0 = s * ROWS_PER_TILE
    for j in range(ROWS_PER_TILE // CHUNK):
        pltpu.sync_copy(rows0, acc.at[pl.ds(r0 + j * CHUNK, CHUNK)])
    plsc.subcore_barrier()

    rows = (rows0, rows1, rows2)
    db = (db0, db1, db2)
    gsem = (gs0, gs1, gs2)
    dsem = (ds0, ds1, ds2)
    ebase = w * EPW

    def gather(q, b):
        pltpu.async_copy(
            table_hbm.at[srcv.at[pl.ds(q * CHUNK, CHUNK)]], rows[b], gsem[b])

    def didx(q, b):
        pltpu.async_copy(
            dst_hbm.at[pl.ds(ebase + q * CHUNK, CHUNK)], db[b], dsem[b])

    def consume(q, b):
        pltpu.make_async_copy(
            table_hbm.at[srcv.at[pl.ds(q * CHUNK, CHUNK)]],
            rows[b], gsem[b]).wait()
        pltpu.make_async_copy(
            dst_hbm.at[pl.ds(ebase + q * CHUNK, CHUNK)], db[b],
            dsem[b]).wait()
        pltpu.sync_copy(rows[b], acc.at[db[b]], add=True)

    for b in range(2):
        didx(b, b)
        gather(b, b)

    def body(p, _):
        for b in range(3):
            q = 3 * p + b
            nb = (b + 2) % 3
            didx(q + 2, nb)
            gather(q + 2, nb)
            consume(q, b)
        return 0
    lax.fori_loop(0, (NITER - 2) // 3, body, 0)

    consume(NITER - 2, (NITER - 2) % 3)
    consume(NITER - 1, (NITER - 1) % 3)

    plsc.subcore_barrier()
    pltpu.sync_copy(
        acc.at[pl.ds(r0, ROWS_PER_TILE)],
        out_hbm.at[pl.ds(c * NPAD + r0, ROWS_PER_TILE)],
    )


def _tc1_body(x_ref, w_ref, d0_ref, d1_ref, xs_ref, dinv_ref):
    deg = d0_ref[...] + d1_ref[...] + 1.0
    dv = lax.rsqrt(deg)
    xw = jnp.dot(x_ref[...], w_ref[...], preferred_element_type=jnp.float32)
    xs_ref[...] = dv * xw
    dinv_ref[...] = dv


def _tc2_body(a0_ref, a1_ref, xs1_ref, dinv_ref, w_ref, b_ref, xs2_ref):
    dv = dinv_ref[...]
    h = dv * (a0_ref[...] + a1_ref[...] + xs1_ref[...]) + b_ref[...]
    h = jnp.maximum(h, 0.0)
    xs2_ref[...] = dv * jnp.dot(h, w_ref[...],
                                preferred_element_type=jnp.float32)


def _tc3_body(a0_ref, a1_ref, xs2_ref, dinv_ref, b_ref, batch_ref,
              pool_ref, cnt_ref):
    i = pl.program_id(0)
    dv = dinv_ref[...]
    out = dv * (a0_ref[...] + a1_ref[...] + xs2_ref[...]) + b_ref[...]

    b_row = batch_ref[0]
    gids = lax.broadcasted_iota(jnp.int32, (G, TCB), 0)
    oht = (gids == b_row).astype(jnp.float32)

    ps = jnp.dot(oht, out, preferred_element_type=jnp.float32)
    cs = jnp.dot(oht, jnp.ones_like(out), preferred_element_type=jnp.float32)

    @pl.when(i == 0)
    def _():
        pool_ref[...] = jnp.zeros_like(pool_ref)
        cnt_ref[...] = jnp.zeros_like(cnt_ref)

    pool_ref[...] += ps
    cnt_ref[...] += cs

    @pl.when(i == TCG - 1)
    def _():
        pool_ref[...] = pool_ref[...] / jnp.maximum(cnt_ref[...], 1.0)


def _rowspec():
    return pl.BlockSpec((TCB, D), lambda i: (i, 0))


def _colspec():
    return pl.BlockSpec((TCB, 1), lambda i: (i, 0))


def _fullspec(shape):
    nd = len(shape)
    return pl.BlockSpec(shape, lambda i: (0,) * nd)


def kernel(x, edge_index, batch, W1, b1, W2, b2):
    src = edge_index[0]
    dst = edge_index[1]

    deg2 = _deg_kernel(dst)
    d0 = deg2[:N].reshape(N, 1)
    d1 = deg2[DEG_PAD:DEG_PAD + N].reshape(N, 1)

    b1r = b1.reshape(1, D)
    b2r = b2.reshape(1, D)
    batch3 = batch.reshape(TCG, 1, TCB)

    xs1, dinv = pl.pallas_call(
        _tc1_body,
        grid=(TCG,),
        in_specs=[_rowspec(), _fullspec((D, D)), _colspec(), _colspec()],
        out_specs=[_rowspec(), _colspec()],
        out_shape=[
            jax.ShapeDtypeStruct((N, D), jnp.float32),
            jax.ShapeDtypeStruct((N, 1), jnp.float32),
        ],
    )(x, W1, d0, d1)

    agg1 = _agg_kernel(xs1, src, dst)

    xs2 = pl.pallas_call(
        _tc2_body,
        grid=(TCG,),
        in_specs=[_rowspec(), _rowspec(), _rowspec(), _colspec(),
                  _fullspec((D, D)), _fullspec((1, D))],
        out_specs=_rowspec(),
        out_shape=jax.ShapeDtypeStruct((N, D), jnp.float32),
    )(agg1[:N], agg1[NPAD:NPAD + N], xs1, dinv, W2, b1r)

    agg2 = _agg_kernel(xs2, src, dst)

    pooled = pl.pallas_call(
        _tc3_body,
        grid=(TCG,),
        in_specs=[_rowspec(), _rowspec(), _rowspec(), _colspec(),
                  _fullspec((1, D)),
                  pl.BlockSpec((1, 1, TCB), lambda i: (i, 0, 0))],
        out_specs=pl.BlockSpec((G, D), lambda i: (0, 0)),
        out_shape=jax.ShapeDtypeStruct((G, D), jnp.float32),
        scratch_shapes=[pltpu.VMEM((G, D), jnp.float32)],
    )(agg2[:N], agg2[NPAD:NPAD + N], xs2, dinv, b2r, batch3)

    return pooled
